# Initial kernel scaffold; baseline (speedup 1.0000x reference)
import jax
import jax.numpy as jnp
from jax.experimental import pallas as pl


def kernel(x, edge_index, edge_attr, batch, W1, b1, W2, b2, W3, b3, W4, b4, Wm1, bm1, Wm2, bm2, Wm3, bm3, Wm4, bm4):
    raise NotImplementedError("write your pallas kernel here")


# speedup vs baseline: 12.6853x; 12.6853x over previous
import functools

import jax
import jax.numpy as jnp
from jax import lax
from jax.experimental import pallas as pl
from jax.experimental.pallas import tpu as pltpu
from jax.experimental.pallas import tpu_sc as plsc

N = 10000
E = 320000
NG = 64
NSC = 2
NTILE = 16
EPT = E // (NSC * NTILE)
K = 80
NCH = EPT // K
STRIPE = 1000

_f32 = jnp.float32
_i32 = jnp.int32


def _mesh():
    return plsc.VectorSubcoreMesh(core_axis_name="c", subcore_axis_name="s")


def _make_edge_pass(Dp):
    @functools.partial(
        pl.kernel,
        mesh=_mesh(),
        out_type=jax.ShapeDtypeStruct((NSC, N, Dp), _f32),
        compiler_params=pltpu.CompilerParams(use_tc_tiling_on_sc=False),
        scratch_types=[
            pltpu.VMEM((K,), _i32),
            pltpu.VMEM((K,), _i32),
            pltpu.VMEM((K, Dp), _f32),
            pltpu.VMEM((STRIPE, Dp), _f32),
            pltpu.VMEM_SHARED((N, Dp), _f32),
            pltpu.SemaphoreType.DMA,
        ],
    )
    def kern(zhat, srci, dsti, out, src_v, dst_v, rows_v, stage_v, acc, sem):
        c = lax.axis_index("c")
        s = lax.axis_index("s")
        wid = c * NTILE + s
        zvec = jnp.zeros((16,), _f32)

        def zrow(r, carry):
            for j in range(Dp // 16):
                stage_v[r, pl.ds(j * 16, 16)] = zvec
            return carry

        lax.fori_loop(0, STRIPE, zrow, 0)

        @pl.when(s < N // STRIPE)
        def _zero():
            pltpu.sync_copy(stage_v, acc.at[pl.ds(s * STRIPE, STRIPE)])

        plsc.subcore_barrier()

        base = wid * EPT

        def body(i, carry):
            off = base + i * K
            pltpu.sync_copy(srci.at[pl.ds(off, K)], src_v)
            pltpu.sync_copy(dsti.at[pl.ds(off, K)], dst_v)
            pltpu.async_copy(zhat.at[src_v], rows_v, sem).wait()
            pltpu.sync_copy(rows_v, acc.at[dst_v], add=True)
            return carry

        lax.fori_loop(0, NCH, body, 0)
        plsc.subcore_barrier()

        @pl.when(s < N // STRIPE)
        def _copy_out():
            pltpu.sync_copy(acc.at[pl.ds(s * STRIPE, STRIPE)], stage_v)
            pltpu.sync_copy(stage_v, out.at[c, pl.ds(s * STRIPE, STRIPE)])

    return kern


def _make_deg_pass():
    @functools.partial(
        pl.kernel,
        mesh=_mesh(),
        out_type=jax.ShapeDtypeStruct((NSC * N,), _f32),
        compiler_params=pltpu.CompilerParams(use_tc_tiling_on_sc=False),
        scratch_types=[
            pltpu.VMEM((K,), _i32),
            pltpu.VMEM((K,), _f32),
            pltpu.VMEM((1024,), _f32),
            pltpu.VMEM_SHARED((N,), _f32),
        ],
    )
    def kern(dsti, out, dst_v, ones_v, zbuf, acc):
        c = lax.axis_index("c")
        s = lax.axis_index("s")
        wid = c * NTILE + s
        onev = jnp.full((16,), 1.0, _f32)
        zvec = jnp.zeros((16,), _f32)
        for j in range(K // 16):
            ones_v[pl.ds(j * 16, 16)] = onev

        def zb(i, carry):
            zbuf[pl.ds(i * 16, 16)] = zvec
            return carry

        lax.fori_loop(0, 64, zb, 0)

        @pl.when(s < 10)
        def _zero():
            pltpu.sync_copy(zbuf.at[pl.ds(0, 1000)], acc.at[pl.ds(s * 1000, 1000)])

        plsc.subcore_barrier()
        base = wid * EPT

        def body(i, carry):
            off = base + i * K
            pltpu.sync_copy(dsti.at[pl.ds(off, K)], dst_v)
            pltpu.sync_copy(ones_v, acc.at[dst_v], add=True)
            return carry

        lax.fori_loop(0, NCH, body, 0)
        plsc.subcore_barrier()

        @pl.when(s < 10)
        def _copy_out():
            pltpu.sync_copy(acc.at[pl.ds(s * 1000, 1000)], zbuf.at[pl.ds(0, 1000)])
            pltpu.sync_copy(zbuf.at[pl.ds(0, 1000)],
                            out.at[pl.ds(c * N + s * 1000, 1000)])

    return kern


def _prep_body(degp_ref, x_ref, w_ref, dinv_ref, zh_ref):
    deg = degp_ref[0] + degp_ref[1] + 1.0
    dinv = lax.rsqrt(deg)
    dinv_ref[...] = dinv
    z = jnp.dot(x_ref[...], w_ref[...], preferred_element_type=_f32)
    zh_ref[...] = z * dinv


def _combine_body(s_ref, zh_ref, dinv_ref, b_ref, w_ref, out_ref):
    h = (s_ref[0] + s_ref[1] + zh_ref[...]) * dinv_ref[...] + b_ref[...]
    h = jnp.maximum(h, 0.0)
    z = jnp.dot(h, w_ref[...], preferred_element_type=_f32)
    out_ref[...] = z * dinv_ref[...]


def _final_body(s_ref, zh_ref, dinv_ref, b_ref, batch_ref,
                wm1, bm1, wm2, bm2, wm3, bm3, wm4, bm4, out_ref):
    h = (s_ref[0] + s_ref[1] + zh_ref[...]) * dinv_ref[...] + b_ref[...]
    h = jnp.maximum(h, 0.0)
    seg = batch_ref[...]
    gid = lax.broadcasted_iota(_i32, (NG, N), 0)
    oh = (gid == seg).astype(_f32)
    sums = jnp.dot(oh, h, preferred_element_type=_f32)
    cnt = jnp.sum(oh, axis=1, keepdims=True)
    g = sums / jnp.maximum(cnt, 1.0)
    g = jnp.maximum(jnp.dot(g, wm1[...], preferred_element_type=_f32) + bm1[...], 0.0)
    g = jnp.maximum(jnp.dot(g, wm2[...], preferred_element_type=_f32) + bm2[...], 0.0)
    g = jnp.maximum(jnp.dot(g, wm3[...], preferred_element_type=_f32) + bm3[...], 0.0)
    out_ref[...] = jnp.dot(g, wm4[...], preferred_element_type=_f32) + bm4[...]


def kernel(x, edge_index, edge_attr, batch,
           W1, b1, W2, b2, W3, b3, W4, b4,
           Wm1, bm1, Wm2, bm2, Wm3, bm3, Wm4, bm4):
    src = edge_index[0]
    dst = edge_index[1]

    W1p = jnp.pad(W1, ((0, 0), (0, 10)))
    W2p = jnp.pad(W2, ((0, 10), (0, 4)))
    W3p = jnp.pad(W3, ((0, 4), (0, 8)))
    W4p = jnp.pad(W4, ((0, 8), (0, 0)))
    b1p = jnp.pad(b1, (0, 10)).reshape(1, 16)
    b2p = jnp.pad(b2, (0, 4)).reshape(1, 16)
    b3p = jnp.pad(b3, (0, 8)).reshape(1, 32)
    b4p = b4.reshape(1, 48)

    degp = _make_deg_pass()(dst)
    degp3 = degp.reshape(NSC, N, 1)

    dinv, zh1 = pl.pallas_call(
        _prep_body,
        out_shape=(jax.ShapeDtypeStruct((N, 1), _f32),
                   jax.ShapeDtypeStruct((N, 16), _f32)),
    )(degp3, x, W1p)

    s1 = _make_edge_pass(16)(zh1, src, dst)
    zh2 = pl.pallas_call(
        _combine_body, out_shape=jax.ShapeDtypeStruct((N, 16), _f32),
    )(s1, zh1, dinv, b1p, W2p)

    s2 = _make_edge_pass(16)(zh2, src, dst)
    zh3 = pl.pallas_call(
        _combine_body, out_shape=jax.ShapeDtypeStruct((N, 32), _f32),
    )(s2, zh2, dinv, b2p, W3p)

    s3 = _make_edge_pass(32)(zh3, src, dst)
    zh4 = pl.pallas_call(
        _combine_body, out_shape=jax.ShapeDtypeStruct((N, 48), _f32),
    )(s3, zh3, dinv, b3p, W4p)

    s4 = _make_edge_pass(48)(zh4, src, dst)
    out = pl.pallas_call(
        _final_body, out_shape=jax.ShapeDtypeStruct((NG, 10), _f32),
    )(s4, zh4, dinv, b4p, batch.reshape(1, N),
      Wm1, bm1.reshape(1, 24), Wm2, bm2.reshape(1, 12),
      Wm3, bm3.reshape(1, 4), Wm4, bm4.reshape(1, 10))
    return out


# speedup vs baseline: 24.7198x; 1.9487x over previous
import functools

import jax
import jax.numpy as jnp
from jax import lax
from jax.experimental import pallas as pl
from jax.experimental.pallas import tpu as pltpu
from jax.experimental.pallas import tpu_sc as plsc

N = 10000
E = 320000
NG = 64
NSC = 2
NTILE = 16
EPT = E // (NSC * NTILE)
K = 100
NCH = EPT // K
NJ = NCH // 2
STRIPE = 1000

_f32 = jnp.float32
_i32 = jnp.int32


def _mesh():
    return plsc.VectorSubcoreMesh(core_axis_name="c", subcore_axis_name="s")


def _make_edge_pass(Dp):
    @functools.partial(
        pl.kernel,
        mesh=_mesh(),
        out_type=jax.ShapeDtypeStruct((NSC, N, Dp), _f32),
        compiler_params=pltpu.CompilerParams(use_tc_tiling_on_sc=False),
        scratch_types=[
            pltpu.VMEM((NCH, K), _i32),
            pltpu.VMEM((NCH, K), _i32),
            pltpu.VMEM((K, Dp), _f32),
            pltpu.VMEM((K, Dp), _f32),
            pltpu.VMEM((STRIPE, Dp), _f32),
            pltpu.VMEM_SHARED((N, Dp), _f32),
            pltpu.SemaphoreType.DMA,
            pltpu.SemaphoreType.DMA,
            pltpu.SemaphoreType.DMA,
            pltpu.SemaphoreType.DMA,
        ],
    )
    def kern(zhat, srcm, dstm, out, srcb, dstb, rows0, rows1, stage_v, acc,
             gs0, gs1, ss0, ss1):
        c = lax.axis_index("c")
        s = lax.axis_index("s")
        wid = c * NTILE + s
        zvec = jnp.zeros((16,), _f32)

        def zrow(r, carry):
            for j in range(Dp // 16):
                stage_v[r, pl.ds(j * 16, 16)] = zvec
            return carry

        lax.fori_loop(0, STRIPE, zrow, 0)

        pltpu.sync_copy(srcm.at[pl.ds(wid * NCH, NCH)], srcb)
        pltpu.sync_copy(dstm.at[pl.ds(wid * NCH, NCH)], dstb)

        @pl.when(s < N // STRIPE)
        def _zero():
            pltpu.sync_copy(stage_v, acc.at[pl.ds(s * STRIPE, STRIPE)])

        plsc.subcore_barrier()

        def gather(i, rbuf, sem):
            return pltpu.async_copy(zhat.at[srcb.at[i]], rbuf, sem)

        def gwait(i, rbuf, sem):
            pltpu.make_async_copy(zhat.at[srcb.at[i]], rbuf, sem).wait()

        def scat(i, rbuf, sem):
            return pltpu.async_copy(rbuf, acc.at[dstb.at[i]], sem, add=True)

        def swait(i, rbuf, sem):
            pltpu.make_async_copy(rbuf, acc.at[dstb.at[i]], sem).wait()

        gather(0, rows0, gs0)

        def body(j, carry):
            i0 = j * 2
            i1 = i0 + 1
            gwait(i0, rows0, gs0)

            @pl.when(j > 0)
            def _():
                swait(i1 - 2, rows1, ss1)

            gather(i1, rows1, gs1)
            scat(i0, rows0, ss0)
            gwait(i1, rows1, gs1)

            @pl.when(j + 1 < NJ)
            def _():
                swait(i0, rows0, ss0)
                gather(i0 + 2, rows0, gs0)

            scat(i1, rows1, ss1)
            return carry

        lax.fori_loop(0, NJ, body, 0)
        swait(NCH - 2, rows0, ss0)
        swait(NCH - 1, rows1, ss1)
        plsc.subcore_barrier()

        @pl.when(s < N // STRIPE)
        def _copy_out():
            pltpu.sync_copy(acc.at[pl.ds(s * STRIPE, STRIPE)], stage_v)
            pltpu.sync_copy(stage_v, out.at[c, pl.ds(s * STRIPE, STRIPE)])

    return kern


def _make_deg_pass():
    DK = 80
    DNCH = EPT // DK

    @functools.partial(
        pl.kernel,
        mesh=_mesh(),
        out_type=jax.ShapeDtypeStruct((NSC * N,), _f32),
        compiler_params=pltpu.CompilerParams(use_tc_tiling_on_sc=False),
        scratch_types=[
            pltpu.VMEM((DK,), _i32),
            pltpu.VMEM((DK,), _f32),
            pltpu.VMEM((1024,), _f32),
            pltpu.VMEM_SHARED((N,), _f32),
        ],
    )
    def kern(dsti, out, dst_v, ones_v, zbuf, acc):
        c = lax.axis_index("c")
        s = lax.axis_index("s")
        wid = c * NTILE + s
        onev = jnp.full((16,), 1.0, _f32)
        zvec = jnp.zeros((16,), _f32)
        for j in range(DK // 16):
            ones_v[pl.ds(j * 16, 16)] = onev

        def zb(i, carry):
            zbuf[pl.ds(i * 16, 16)] = zvec
            return carry

        lax.fori_loop(0, 64, zb, 0)

        @pl.when(s < 10)
        def _zero():
            pltpu.sync_copy(zbuf.at[pl.ds(0, 1000)], acc.at[pl.ds(s * 1000, 1000)])

        plsc.subcore_barrier()
        base = wid * EPT

        def body(i, carry):
            off = base + i * DK
            pltpu.sync_copy(dsti.at[pl.ds(off, DK)], dst_v)
            pltpu.sync_copy(ones_v, acc.at[dst_v], add=True)
            return carry

        lax.fori_loop(0, DNCH, body, 0)
        plsc.subcore_barrier()

        @pl.when(s < 10)
        def _copy_out():
            pltpu.sync_copy(acc.at[pl.ds(s * 1000, 1000)], zbuf.at[pl.ds(0, 1000)])
            pltpu.sync_copy(zbuf.at[pl.ds(0, 1000)],
                            out.at[pl.ds(c * N + s * 1000, 1000)])

    return kern


def _prep_body(degp_ref, x_ref, w_ref, dinv_ref, zh_ref):
    deg = degp_ref[0] + degp_ref[1] + 1.0
    dinv = lax.rsqrt(deg)
    dinv_ref[...] = dinv
    z = jnp.dot(x_ref[...], w_ref[...], preferred_element_type=_f32)
    zh_ref[...] = z * dinv


def _combine_body(s_ref, zh_ref, dinv_ref, b_ref, w_ref, out_ref):
    h = (s_ref[0] + s_ref[1] + zh_ref[...]) * dinv_ref[...] + b_ref[...]
    h = jnp.maximum(h, 0.0)
    z = jnp.dot(h, w_ref[...], preferred_element_type=_f32)
    out_ref[...] = z * dinv_ref[...]


def _final_body(s_ref, zh_ref, dinv_ref, b_ref, batch_ref,
                wm1, bm1, wm2, bm2, wm3, bm3, wm4, bm4, out_ref):
    h = (s_ref[0] + s_ref[1] + zh_ref[...]) * dinv_ref[...] + b_ref[...]
    h = jnp.maximum(h, 0.0)
    seg = batch_ref[...]
    gid = lax.broadcasted_iota(_i32, (NG, N), 0)
    oh = (gid == seg).astype(_f32)
    sums = jnp.dot(oh, h, preferred_element_type=_f32)
    cnt = jnp.sum(oh, axis=1, keepdims=True)
    g = sums / jnp.maximum(cnt, 1.0)
    g = jnp.maximum(jnp.dot(g, wm1[...], preferred_element_type=_f32) + bm1[...], 0.0)
    g = jnp.maximum(jnp.dot(g, wm2[...], preferred_element_type=_f32) + bm2[...], 0.0)
    g = jnp.maximum(jnp.dot(g, wm3[...], preferred_element_type=_f32) + bm3[...], 0.0)
    out_ref[...] = jnp.dot(g, wm4[...], preferred_element_type=_f32) + bm4[...]


def kernel(x, edge_index, edge_attr, batch,
           W1, b1, W2, b2, W3, b3, W4, b4,
           Wm1, bm1, Wm2, bm2, Wm3, bm3, Wm4, bm4):
    src = edge_index[0]
    dst = edge_index[1]
    srcm = src.reshape(E // K, K)
    dstm = dst.reshape(E // K, K)

    W1p = jnp.pad(W1, ((0, 0), (0, 10)))
    W2p = jnp.pad(W2, ((0, 10), (0, 4)))
    W3p = jnp.pad(W3, ((0, 4), (0, 8)))
    W4p = jnp.pad(W4, ((0, 8), (0, 0)))
    b1p = jnp.pad(b1, (0, 10)).reshape(1, 16)
    b2p = jnp.pad(b2, (0, 4)).reshape(1, 16)
    b3p = jnp.pad(b3, (0, 8)).reshape(1, 32)
    b4p = b4.reshape(1, 48)

    degp = _make_deg_pass()(dst)
    degp3 = degp.reshape(NSC, N, 1)

    dinv, zh1 = pl.pallas_call(
        _prep_body,
        out_shape=(jax.ShapeDtypeStruct((N, 1), _f32),
                   jax.ShapeDtypeStruct((N, 16), _f32)),
    )(degp3, x, W1p)

    s1 = _make_edge_pass(16)(zh1, srcm, dstm)
    zh2 = pl.pallas_call(
        _combine_body, out_shape=jax.ShapeDtypeStruct((N, 16), _f32),
    )(s1, zh1, dinv, b1p, W2p)

    s2 = _make_edge_pass(16)(zh2, srcm, dstm)
    zh3 = pl.pallas_call(
        _combine_body, out_shape=jax.ShapeDtypeStruct((N, 32), _f32),
    )(s2, zh2, dinv, b2p, W3p)

    s3 = _make_edge_pass(32)(zh3, srcm, dstm)
    zh4 = pl.pallas_call(
        _combine_body, out_shape=jax.ShapeDtypeStruct((N, 48), _f32),
    )(s3, zh3, dinv, b3p, W4p)

    s4 = _make_edge_pass(48)(zh4, srcm, dstm)
    out = pl.pallas_call(
        _final_body, out_shape=jax.ShapeDtypeStruct((NG, 10), _f32),
    )(s4, zh4, dinv, b4p, batch.reshape(1, N),
      Wm1, bm1.reshape(1, 24), Wm2, bm2.reshape(1, 12),
      Wm3, bm3.reshape(1, 4), Wm4, bm4.reshape(1, 10))
    return out


# speedup vs baseline: 46.7610x; 1.8916x over previous
import functools

import jax
import jax.numpy as jnp
from jax import lax
from jax.experimental import pallas as pl
from jax.experimental.pallas import tpu as pltpu
from jax.experimental.pallas import tpu_sc as plsc

N = 10000
E = 320000
NG = 64
NSC = 2
NTILE = 16
EPT = E // (NSC * NTILE)
K = 125
NCH = EPT // K
NBUF = 8
NGRP = NCH // NBUF
STRIPE = 1000
SROWS = 500

_f32 = jnp.float32
_i32 = jnp.int32


def _mesh():
    return plsc.VectorSubcoreMesh(core_axis_name="c", subcore_axis_name="s")


def _make_edge_pass(Dp):
    @functools.partial(
        pl.kernel,
        mesh=_mesh(),
        out_type=jax.ShapeDtypeStruct((NSC, N, Dp), _f32),
        compiler_params=pltpu.CompilerParams(use_tc_tiling_on_sc=False),
        scratch_types=[
            pltpu.VMEM((NCH, K), _i32),
            pltpu.VMEM((NCH, K), _i32),
            [pltpu.VMEM((K, Dp), _f32) for _ in range(NBUF)],
            pltpu.VMEM((SROWS, Dp), _f32),
            pltpu.VMEM_SHARED((N, Dp), _f32),
            [pltpu.SemaphoreType.DMA for _ in range(NBUF)],
            [pltpu.SemaphoreType.DMA for _ in range(NBUF)],
        ],
    )
    def kern(zhat, srcm, dstm, out, srcb, dstb, rows, stage_v, acc, gsem, ssem):
        c = lax.axis_index("c")
        s = lax.axis_index("s")
        wid = c * NTILE + s
        zvec = jnp.zeros((16,), _f32)

        def zrow(r, carry):
            for j in range(Dp // 16):
                stage_v[r, pl.ds(j * 16, 16)] = zvec
            return carry

        lax.fori_loop(0, SROWS, zrow, 0)

        pltpu.sync_copy(srcm.at[pl.ds(wid * NCH, NCH)], srcb)
        pltpu.sync_copy(dstm.at[pl.ds(wid * NCH, NCH)], dstb)

        @pl.when(s < N // STRIPE)
        def _zero():
            pltpu.sync_copy(stage_v, acc.at[pl.ds(s * STRIPE, SROWS)])
            pltpu.sync_copy(stage_v, acc.at[pl.ds(s * STRIPE + SROWS, SROWS)])

        plsc.subcore_barrier()

        def gather(i, b):
            return pltpu.async_copy(zhat.at[srcb.at[i]], rows[b], gsem[b])

        def gwait(i, b):
            pltpu.make_async_copy(zhat.at[srcb.at[i]], rows[b], gsem[b]).wait()

        def scat(i, b):
            return pltpu.async_copy(rows[b], acc.at[dstb.at[i]], ssem[b], add=True)

        def swait(i, b):
            pltpu.make_async_copy(rows[b], acc.at[dstb.at[i]], ssem[b]).wait()

        for b in range(NBUF):
            gather(b, b)

        def body(j, carry):
            ibase = j * NBUF
            for b in range(NBUF):
                gwait(ibase + b, b)
                scat(ibase + b, b)

            @pl.when(j + 1 < NGRP)
            def _refill():
                for b in range(NBUF):
                    swait(ibase + b, b)
                    gather(ibase + NBUF + b, b)

            return carry

        lax.fori_loop(0, NGRP, body, 0)
        for b in range(NBUF):
            swait(NCH - NBUF + b, b)
        plsc.subcore_barrier()

        @pl.when(s < N // STRIPE)
        def _copy_out():
            pltpu.sync_copy(acc.at[pl.ds(s * STRIPE, SROWS)], stage_v)
            pltpu.sync_copy(stage_v, out.at[c, pl.ds(s * STRIPE, SROWS)])
            pltpu.sync_copy(acc.at[pl.ds(s * STRIPE + SROWS, SROWS)], stage_v)
            pltpu.sync_copy(stage_v, out.at[c, pl.ds(s * STRIPE + SROWS, SROWS)])

    return kern


def _make_deg_pass():
    @functools.partial(
        pl.kernel,
        mesh=_mesh(),
        out_type=jax.ShapeDtypeStruct((NSC * N,), _f32),
        compiler_params=pltpu.CompilerParams(use_tc_tiling_on_sc=False,
                                             needs_layout_passes=False),
        scratch_types=[
            pltpu.VMEM((EPT,), _i32),
            pltpu.VMEM((N,), _f32),
            pltpu.VMEM((2000,), _f32),
            pltpu.VMEM_SHARED((NTILE, N), _f32),
        ],
    )
    def kern(dsti, out, dstb, dacc, tmp, acc2):
        c = lax.axis_index("c")
        s = lax.axis_index("s")
        wid = c * NTILE + s
        pltpu.sync_copy(dsti.at[pl.ds(wid * EPT, EPT)], dstb)
        onev = jnp.full((16,), 1.0, _f32)
        zvec = jnp.zeros((16,), _f32)

        def zb(i, carry):
            dacc[pl.ds(i * 16, 16)] = zvec
            return carry

        lax.fori_loop(0, N // 16, zb, 0)

        def body(i, carry):
            idx = dstb[pl.ds(i * 16, 16)]
            plsc.addupdate_scatter(dacc, [idx], onev)
            return carry

        lax.fori_loop(0, EPT // 16, body, 0)
        pltpu.sync_copy(dacc, acc2.at[s])
        plsc.subcore_barrier()

        @pl.when(s < 5)
        def _merge():
            base = s * 2000
            pltpu.sync_copy(acc2.at[0, pl.ds(base, 2000)], dacc.at[pl.ds(0, 2000)])
            for t in range(1, NTILE):
                pltpu.sync_copy(acc2.at[t, pl.ds(base, 2000)], tmp)

                def madd(i, carry):
                    dacc[pl.ds(i * 16, 16)] = (dacc[pl.ds(i * 16, 16)]
                                               + tmp[pl.ds(i * 16, 16)])
                    return carry

                lax.fori_loop(0, 125, madd, 0)
            pltpu.sync_copy(dacc.at[pl.ds(0, 2000)],
                            out.at[pl.ds(c * N + base, 2000)])

    return kern


def _prep_body(degp_ref, x_ref, w_ref, dinv_ref, zh_ref):
    deg = degp_ref[0] + degp_ref[1] + 1.0
    dinv = lax.rsqrt(deg)
    dinv_ref[...] = dinv
    z = jnp.dot(x_ref[...], w_ref[...], preferred_element_type=_f32)
    zh_ref[...] = z * dinv


def _combine_body(s_ref, zh_ref, dinv_ref, b_ref, w_ref, out_ref):
    h = (s_ref[0] + s_ref[1] + zh_ref[...]) * dinv_ref[...] + b_ref[...]
    h = jnp.maximum(h, 0.0)
    z = jnp.dot(h, w_ref[...], preferred_element_type=_f32)
    out_ref[...] = z * dinv_ref[...]


def _final_body(s_ref, zh_ref, dinv_ref, b_ref, batch_ref,
                wm1, bm1, wm2, bm2, wm3, bm3, wm4, bm4, out_ref):
    h = (s_ref[0] + s_ref[1] + zh_ref[...]) * dinv_ref[...] + b_ref[...]
    h = jnp.maximum(h, 0.0)
    seg = batch_ref[...]
    gid = lax.broadcasted_iota(_i32, (NG, N), 0)
    oh = (gid == seg).astype(_f32)
    sums = jnp.dot(oh, h, preferred_element_type=_f32)
    cnt = jnp.sum(oh, axis=1, keepdims=True)
    g = sums / jnp.maximum(cnt, 1.0)
    g = jnp.maximum(jnp.dot(g, wm1[...], preferred_element_type=_f32) + bm1[...], 0.0)
    g = jnp.maximum(jnp.dot(g, wm2[...], preferred_element_type=_f32) + bm2[...], 0.0)
    g = jnp.maximum(jnp.dot(g, wm3[...], preferred_element_type=_f32) + bm3[...], 0.0)
    out_ref[...] = jnp.dot(g, wm4[...], preferred_element_type=_f32) + bm4[...]


def kernel(x, edge_index, edge_attr, batch,
           W1, b1, W2, b2, W3, b3, W4, b4,
           Wm1, bm1, Wm2, bm2, Wm3, bm3, Wm4, bm4):
    src = edge_index[0]
    dst = edge_index[1]
    srcm = src.reshape(E // K, K)
    dstm = dst.reshape(E // K, K)

    W1p = jnp.pad(W1, ((0, 0), (0, 10)))
    W2p = jnp.pad(W2, ((0, 10), (0, 4)))
    W3p = jnp.pad(W3, ((0, 4), (0, 8)))
    W4p = jnp.pad(W4, ((0, 8), (0, 0)))
    b1p = jnp.pad(b1, (0, 10)).reshape(1, 16)
    b2p = jnp.pad(b2, (0, 4)).reshape(1, 16)
    b3p = jnp.pad(b3, (0, 8)).reshape(1, 32)
    b4p = b4.reshape(1, 48)

    degp = _make_deg_pass()(dst)
    degp3 = degp.reshape(NSC, N, 1)

    dinv, zh1 = pl.pallas_call(
        _prep_body,
        out_shape=(jax.ShapeDtypeStruct((N, 1), _f32),
                   jax.ShapeDtypeStruct((N, 16), _f32)),
    )(degp3, x, W1p)

    s1 = _make_edge_pass(16)(zh1, srcm, dstm)
    zh2 = pl.pallas_call(
        _combine_body, out_shape=jax.ShapeDtypeStruct((N, 16), _f32),
    )(s1, zh1, dinv, b1p, W2p)

    s2 = _make_edge_pass(16)(zh2, srcm, dstm)
    zh3 = pl.pallas_call(
        _combine_body, out_shape=jax.ShapeDtypeStruct((N, 32), _f32),
    )(s2, zh2, dinv, b2p, W3p)

    s3 = _make_edge_pass(32)(zh3, srcm, dstm)
    zh4 = pl.pallas_call(
        _combine_body, out_shape=jax.ShapeDtypeStruct((N, 48), _f32),
    )(s3, zh3, dinv, b3p, W4p)

    s4 = _make_edge_pass(48)(zh4, srcm, dstm)
    out = pl.pallas_call(
        _final_body, out_shape=jax.ShapeDtypeStruct((NG, 10), _f32),
    )(s4, zh4, dinv, b4p, batch.reshape(1, N),
      Wm1, bm1.reshape(1, 24), Wm2, bm2.reshape(1, 12),
      Wm3, bm3.reshape(1, 4), Wm4, bm4.reshape(1, 10))
    return out


# speedup vs baseline: 59.4217x; 1.2708x over previous
import functools

import jax
import jax.numpy as jnp
import numpy as np
from jax import lax
from jax.experimental import pallas as pl
from jax.experimental.pallas import tpu as pltpu
from jax.experimental.pallas import tpu_sc as plsc

N = 10000
E = 320000
NG = 64
NSC = 2
NTILE = 16
EPT = E // (NSC * NTILE)
K = 125
NCH = EPT // K
NBUF = 8
NGRP = NCH // NBUF
STRIPE = 1000
SROWS = 500

_f32 = jnp.float32
_i32 = jnp.int32


def _mesh():
    return plsc.VectorSubcoreMesh(core_axis_name="c", subcore_axis_name="s")


def _make_edge_pass(Dp):
    @functools.partial(
        pl.kernel,
        mesh=_mesh(),
        out_type=jax.ShapeDtypeStruct((NSC, N, Dp), _f32),
        compiler_params=pltpu.CompilerParams(use_tc_tiling_on_sc=False),
        scratch_types=[
            pltpu.VMEM((NCH, K), _i32),
            pltpu.VMEM((NCH, K), _i32),
            [pltpu.VMEM((K, Dp), _f32) for _ in range(NBUF)],
            pltpu.VMEM((SROWS, Dp), _f32),
            pltpu.VMEM_SHARED((N, Dp), _f32),
            [pltpu.SemaphoreType.DMA for _ in range(NBUF)],
            [pltpu.SemaphoreType.DMA for _ in range(NBUF)],
        ],
    )
    def kern(zhat, srcm, dstm, out, srcb, dstb, rows, stage_v, acc, gsem, ssem):
        c = lax.axis_index("c")
        s = lax.axis_index("s")
        wid = c * NTILE + s
        zvec = jnp.zeros((16,), _f32)

        def zrow(r, carry):
            for j in range(Dp // 16):
                stage_v[r, pl.ds(j * 16, 16)] = zvec
            return carry

        lax.fori_loop(0, SROWS, zrow, 0)

        pltpu.sync_copy(srcm.at[pl.ds(wid * NCH, NCH)], srcb)
        pltpu.sync_copy(dstm.at[pl.ds(wid * NCH, NCH)], dstb)

        @pl.when(s < N // STRIPE)
        def _zero():
            pltpu.sync_copy(stage_v, acc.at[pl.ds(s * STRIPE, SROWS)])
            pltpu.sync_copy(stage_v, acc.at[pl.ds(s * STRIPE + SROWS, SROWS)])

        plsc.subcore_barrier()

        def gather(i, b):
            return pltpu.async_copy(zhat.at[srcb.at[i]], rows[b], gsem[b])

        def gwait(i, b):
            pltpu.make_async_copy(zhat.at[srcb.at[i]], rows[b], gsem[b]).wait()

        def scat(i, b):
            return pltpu.async_copy(rows[b], acc.at[dstb.at[i]], ssem[b], add=True)

        def swait(i, b):
            pltpu.make_async_copy(rows[b], acc.at[dstb.at[i]], ssem[b]).wait()

        for b in range(NBUF):
            gather(b, b)

        def body(j, carry):
            ibase = j * NBUF
            for b in range(NBUF):
                gwait(ibase + b, b)
                scat(ibase + b, b)

            @pl.when(j + 1 < NGRP)
            def _refill():
                for b in range(NBUF):
                    swait(ibase + b, b)
                    gather(ibase + NBUF + b, b)

            return carry

        lax.fori_loop(0, NGRP, body, 0)
        for b in range(NBUF):
            swait(NCH - NBUF + b, b)
        plsc.subcore_barrier()

        @pl.when(s < N // STRIPE)
        def _copy_out():
            pltpu.sync_copy(acc.at[pl.ds(s * STRIPE, SROWS)], stage_v)
            pltpu.sync_copy(stage_v, out.at[c, pl.ds(s * STRIPE, SROWS)])
            pltpu.sync_copy(acc.at[pl.ds(s * STRIPE + SROWS, SROWS)], stage_v)
            pltpu.sync_copy(stage_v, out.at[c, pl.ds(s * STRIPE + SROWS, SROWS)])

    return kern


def _make_deg_pass():
    @functools.partial(
        pl.kernel,
        mesh=_mesh(),
        out_type=jax.ShapeDtypeStruct((NSC * N,), _f32),
        compiler_params=pltpu.CompilerParams(use_tc_tiling_on_sc=False,
                                             needs_layout_passes=False),
        scratch_types=[
            pltpu.VMEM((EPT,), _i32),
            pltpu.VMEM((N,), _f32),
            pltpu.VMEM((2000,), _f32),
            pltpu.VMEM_SHARED((NTILE, N), _f32),
        ],
    )
    def kern(dsti, out, dstb, dacc, tmp, acc2):
        c = lax.axis_index("c")
        s = lax.axis_index("s")
        wid = c * NTILE + s
        pltpu.sync_copy(dsti.at[pl.ds(wid * EPT, EPT)], dstb)
        onev = jnp.full((16,), 1.0, _f32)
        zvec = jnp.zeros((16,), _f32)

        def zb(i, carry):
            dacc[pl.ds(i * 16, 16)] = zvec
            return carry

        lax.fori_loop(0, N // 16, zb, 0)

        def body(i, carry):
            idx = dstb[pl.ds(i * 16, 16)]
            plsc.addupdate_scatter(dacc, [idx], onev)
            return carry

        lax.fori_loop(0, EPT // 16, body, 0)
        pltpu.sync_copy(dacc, acc2.at[s])
        plsc.subcore_barrier()

        @pl.when(s < 5)
        def _merge():
            base = s * 2000
            pltpu.sync_copy(acc2.at[0, pl.ds(base, 2000)], dacc.at[pl.ds(0, 2000)])
            for t in range(1, NTILE):
                pltpu.sync_copy(acc2.at[t, pl.ds(base, 2000)], tmp)

                def madd(i, carry):
                    dacc[pl.ds(i * 16, 16)] = (dacc[pl.ds(i * 16, 16)]
                                               + tmp[pl.ds(i * 16, 16)])
                    return carry

                lax.fori_loop(0, 125, madd, 0)
            pltpu.sync_copy(dacc.at[pl.ds(0, 2000)],
                            out.at[pl.ds(c * N + base, 2000)])

    return kern


R = N // 8


def _prep_body(degp_ref, x8_ref, w1bd_ref, s16_ref, s32_ref, s48_ref,
               zh1_ref, dp16_ref, dp32_ref, dp48_ref):
    deg = degp_ref[0] + degp_ref[1] + 1.0
    dinv8 = lax.rsqrt(deg)
    dp16 = jnp.dot(dinv8, s16_ref[...], preferred_element_type=_f32)
    dp16_ref[...] = dp16
    dp32_ref[...] = jnp.dot(dinv8, s32_ref[...], preferred_element_type=_f32)
    dp48_ref[...] = jnp.dot(dinv8, s48_ref[...], preferred_element_type=_f32)
    z = jnp.dot(x8_ref[...], w1bd_ref[...], preferred_element_type=_f32)
    zh1_ref[...] = z * dp16


def _combine_body(s_ref, zh_ref, dpin_ref, b_ref, w_ref, dpout_ref, out_ref):
    h = (s_ref[0] + s_ref[1] + zh_ref[...]) * dpin_ref[...] + b_ref[...]
    h = jnp.maximum(h, 0.0)
    z = jnp.dot(h, w_ref[...], preferred_element_type=_f32)
    out_ref[...] = z * dpout_ref[...]


def _final_body(s_ref, zh_ref, dp_ref, b_ref, batch_ref,
                wm1, bm1, wm2, bm2, wm3, bm3, wm4, bm4, out_ref):
    h = (s_ref[0] + s_ref[1] + zh_ref[...]) * dp_ref[...] + b_ref[...]
    h = jnp.maximum(h, 0.0)
    batchp = batch_ref[...]
    gid = lax.broadcasted_iota(_i32, (R, NG), 1)
    ones_col = jnp.ones((R, 1), _f32)
    acc = jnp.zeros((NG, 49), _f32)
    for b in range(8):
        oht = (batchp[:, b:b + 1] == gid).astype(_f32)
        haug = jnp.concatenate([h[:, 48 * b:48 * b + 48], ones_col], axis=1)
        acc = acc + lax.dot_general(oht, haug, (((0,), (0,)), ((), ())),
                                    preferred_element_type=_f32)
    g = acc[:, :48] / jnp.maximum(acc[:, 48:49], 1.0)
    g = jnp.maximum(jnp.dot(g, wm1[...], preferred_element_type=_f32) + bm1[...], 0.0)
    g = jnp.maximum(jnp.dot(g, wm2[...], preferred_element_type=_f32) + bm2[...], 0.0)
    g = jnp.maximum(jnp.dot(g, wm3[...], preferred_element_type=_f32) + bm3[...], 0.0)
    out_ref[...] = jnp.dot(g, wm4[...], preferred_element_type=_f32) + bm4[...]


def _blockdiag8(w):
    di, do = w.shape
    out = jnp.zeros((8 * di, 8 * do), w.dtype)
    for k in range(8):
        out = out.at[k * di:(k + 1) * di, k * do:(k + 1) * do].set(w)
    return out


def _sel8(width):
    c = np.arange(8 * width) // width
    return jnp.asarray((c[None, :] == np.arange(8)[:, None]).astype(np.float32))


def kernel(x, edge_index, edge_attr, batch,
           W1, b1, W2, b2, W3, b3, W4, b4,
           Wm1, bm1, Wm2, bm2, Wm3, bm3, Wm4, bm4):
    src = edge_index[0]
    dst = edge_index[1]
    srcm = src.reshape(E // K, K)
    dstm = dst.reshape(E // K, K)

    W1bd = _blockdiag8(jnp.pad(W1, ((0, 0), (0, 10))))
    W2bd = _blockdiag8(jnp.pad(W2, ((0, 10), (0, 4))))
    W3bd = _blockdiag8(jnp.pad(W3, ((0, 4), (0, 8))))
    W4bd = _blockdiag8(jnp.pad(W4, ((0, 8), (0, 0))))
    b1P = jnp.tile(jnp.pad(b1, (0, 10)), 8).reshape(1, 128)
    b2P = jnp.tile(jnp.pad(b2, (0, 4)), 8).reshape(1, 128)
    b3P = jnp.tile(jnp.pad(b3, (0, 8)), 8).reshape(1, 256)
    b4P = jnp.tile(b4, 8).reshape(1, 384)
    sel16, sel32, sel48 = _sel8(16), _sel8(32), _sel8(48)
    x8 = x.reshape(R, 1024)
    batchp = batch.reshape(R, 8)

    degp = _make_deg_pass()(dst)
    degp3 = degp.reshape(NSC, R, 8)

    zh1p, dp16, dp32, dp48 = pl.pallas_call(
        _prep_body,
        out_shape=(jax.ShapeDtypeStruct((R, 128), _f32),
                   jax.ShapeDtypeStruct((R, 128), _f32),
                   jax.ShapeDtypeStruct((R, 256), _f32),
                   jax.ShapeDtypeStruct((R, 384), _f32)),
    )(degp3, x8, W1bd, sel16, sel32, sel48)

    s1 = _make_edge_pass(16)(zh1p.reshape(N, 16), srcm, dstm)
    zh2p = pl.pallas_call(
        _combine_body, out_shape=jax.ShapeDtypeStruct((R, 128), _f32),
    )(s1.reshape(NSC, R, 128), zh1p, dp16, b1P, W2bd, dp16)

    s2 = _make_edge_pass(16)(zh2p.reshape(N, 16), srcm, dstm)
    zh3p = pl.pallas_call(
        _combine_body, out_shape=jax.ShapeDtypeStruct((R, 256), _f32),
    )(s2.reshape(NSC, R, 128), zh2p, dp16, b2P, W3bd, dp32)

    s3 = _make_edge_pass(32)(zh3p.reshape(N, 32), srcm, dstm)
    zh4p = pl.pallas_call(
        _combine_body, out_shape=jax.ShapeDtypeStruct((R, 384), _f32),
    )(s3.reshape(NSC, R, 256), zh3p, dp32, b3P, W4bd, dp48)

    s4 = _make_edge_pass(48)(zh4p.reshape(N, 48), srcm, dstm)
    out = pl.pallas_call(
        _final_body, out_shape=jax.ShapeDtypeStruct((NG, 10), _f32),
    )(s4.reshape(NSC, R, 384), zh4p, dp48, b4P, batchp,
      Wm1, bm1.reshape(1, 24), Wm2, bm2.reshape(1, 12),
      Wm3, bm3.reshape(1, 4), Wm4, bm4.reshape(1, 10))
    return out
